# baseline (speedup 1.0000x reference)
import functools

import jax
import jax.numpy as jnp
from jax import lax
from jax.experimental import pallas as pl
from jax.experimental.pallas import tpu as pltpu
from jax.experimental.pallas import tpu_sc as plsc

B = 16384
DIM = 64
NC = 2
NS = 16
NW = NC * NS
BPW = B // NW
CH = 128
NCH = BPW // CH
GRP = CH // 16


def _sc_body(h_hbm, r_hbm, t_hbm, ecat_hbm, rcat_hbm, out_hbm,
             hidx, ridx, tidx, hrow, rrow, trow, tmp, outv, sem):
    wid = lax.axis_index("s") * NC + lax.axis_index("c")
    base = wid * BPW
    for c in range(NCH):
        off = base + c * CH
        pltpu.sync_copy(h_hbm.at[pl.ds(off, CH)], hidx)
        pltpu.sync_copy(r_hbm.at[pl.ds(off, CH)], ridx)
        pltpu.sync_copy(t_hbm.at[pl.ds(off, CH)], tidx)
        cps = [
            pltpu.async_copy(ecat_hbm.at[hidx], hrow, sem),
            pltpu.async_copy(rcat_hbm.at[ridx], rrow, sem),
            pltpu.async_copy(ecat_hbm.at[tidx], trow, sem),
        ]
        for cp in cps:
            cp.wait()
        lanes = lax.broadcasted_iota(jnp.int32, (16,), 0)

        def group(g, _, c=c):
            gsl = pl.ds(g * 16, 16)
            for e in range(16):
                i = g * 16 + e
                q = jnp.zeros((16,), jnp.float32)
                for k in range(DIM // 16):
                    re_sl = pl.ds(k * 16, 16)
                    im_sl = pl.ds(DIM + k * 16, 16)
                    a = hrow[i, re_sl]
                    b = hrow[i, im_sl]
                    cr = rrow[i, re_sl]
                    ci = rrow[i, im_sl]
                    dr = trow[i, re_sl]
                    di = trow[i, im_sl]
                    q = q + dr * (a * cr - b * ci) + di * (a * ci + b * cr)
                plsc.store_scatter(tmp, [lanes * 16 + e], q)
            s = tmp[pl.ds(0, 16)]
            for l in range(1, 16):
                s = s + tmp[pl.ds(l * 16, 16)]
            s = 1.0 / (1.0 + jnp.exp(-s))
            outv[pl.ds(c * CH + g * 16, 16)] = s
            return 0

        lax.fori_loop(0, GRP, group, 0)
    pltpu.sync_copy(outv, out_hbm.at[pl.ds(base, BPW)])


@jax.jit
def _run(h, r, t, ecat, rcat):
    mesh = plsc.VectorSubcoreMesh(core_axis_name="c", subcore_axis_name="s")
    gather_buf = pltpu.VMEM((CH, 2 * DIM), jnp.float32)
    kern = functools.partial(
        pl.kernel,
        mesh=mesh,
        compiler_params=pltpu.CompilerParams(needs_layout_passes=False),
        out_type=jax.ShapeDtypeStruct((B,), jnp.float32),
        scratch_types=[
            pltpu.VMEM((CH,), jnp.int32),
            pltpu.VMEM((CH,), jnp.int32),
            pltpu.VMEM((CH,), jnp.int32),
            gather_buf,
            gather_buf,
            gather_buf,
            pltpu.VMEM((256,), jnp.float32),
            pltpu.VMEM((BPW,), jnp.float32),
            pltpu.SemaphoreType.DMA,
        ],
    )(_sc_body)
    return kern(h, r, t, ecat, rcat)


def kernel(h, r, t, batch_size, emb_e_real, emb_e_img, emb_rel_real,
           emb_rel_img):
    ecat = jnp.zeros((emb_e_real.shape[0], 2 * DIM), jnp.float32)
    ecat = lax.dynamic_update_slice(ecat, emb_e_real, (0, 0))
    ecat = lax.dynamic_update_slice(ecat, emb_e_img, (0, DIM))
    rcat = jnp.zeros((emb_rel_real.shape[0], 2 * DIM), jnp.float32)
    rcat = lax.dynamic_update_slice(rcat, emb_rel_real, (0, 0))
    rcat = lax.dynamic_update_slice(rcat, emb_rel_img, (0, DIM))
    score = _run(h, r, t, ecat, rcat)
    return score[:8192], score[8192:]


# speedup vs baseline: 3.9117x; 3.9117x over previous
import functools

import jax
import jax.numpy as jnp
from jax import lax
from jax.experimental import pallas as pl
from jax.experimental.pallas import tpu as pltpu
from jax.experimental.pallas import tpu_sc as plsc

B = 16384
DIM = 64
NC = 2
NS = 16
NW = NC * NS
BPW = B // NW
CH = 128
NCH = BPW // CH
GRP = CH // 16


def _sc_body(h_hbm, r_hbm, t_hbm, ecat_hbm, rcat_hbm, out_hbm,
             hidx, ridx, tidx, hrow, rrow, trow, tmp, outv, sem):
    wid = lax.axis_index("s") * NC + lax.axis_index("c")
    base = wid * BPW
    for c in range(NCH):
        off = base + c * CH
        pltpu.sync_copy(h_hbm.at[pl.ds(off, CH)], hidx)
        pltpu.sync_copy(r_hbm.at[pl.ds(off, CH)], ridx)
        pltpu.sync_copy(t_hbm.at[pl.ds(off, CH)], tidx)
        cps = [
            pltpu.async_copy(ecat_hbm.at[hidx], hrow, sem),
            pltpu.async_copy(rcat_hbm.at[ridx], rrow, sem),
            pltpu.async_copy(ecat_hbm.at[tidx], trow, sem),
        ]
        for cp in cps:
            cp.wait()
        lanes = lax.broadcasted_iota(jnp.int32, (16,), 0)

        def group(g, _, c=c):
            gsl = pl.ds(g * 16, 16)
            for e in range(16):
                i = g * 16 + e
                q = jnp.zeros((16,), jnp.float32)
                for k in range(DIM // 16):
                    re_sl = pl.ds(k * 16, 16)
                    im_sl = pl.ds(DIM + k * 16, 16)
                    a = hrow[i, re_sl]
                    b = hrow[i, im_sl]
                    cr = rrow[i, re_sl]
                    ci = rrow[i, im_sl]
                    dr = trow[i, re_sl]
                    di = trow[i, im_sl]
                    q = q + dr * (a * cr - b * ci) + di * (a * ci + b * cr)
                plsc.store_scatter(tmp, [lanes * 16 + e], q)
            s = tmp[pl.ds(0, 16)]
            for l in range(1, 16):
                s = s + tmp[pl.ds(l * 16, 16)]
            s = 1.0 / (1.0 + jnp.exp(-s))
            outv[pl.ds(c * CH + g * 16, 16)] = s
            return 0

        lax.fori_loop(0, GRP, group, 0)
    pltpu.sync_copy(outv, out_hbm.at[pl.ds(base, BPW)])


@jax.jit
def _run(h, r, t, ecat, rcat):
    mesh = plsc.VectorSubcoreMesh(core_axis_name="c", subcore_axis_name="s")
    gather_buf = pltpu.VMEM((CH, 2 * DIM), jnp.float32)
    kern = functools.partial(
        pl.kernel,
        mesh=mesh,
        compiler_params=pltpu.CompilerParams(needs_layout_passes=False),
        out_type=jax.ShapeDtypeStruct((B,), jnp.float32),
        scratch_types=[
            pltpu.VMEM((CH,), jnp.int32),
            pltpu.VMEM((CH,), jnp.int32),
            pltpu.VMEM((CH,), jnp.int32),
            gather_buf,
            gather_buf,
            gather_buf,
            pltpu.VMEM((256,), jnp.float32),
            pltpu.VMEM((BPW,), jnp.float32),
            pltpu.SemaphoreType.DMA,
        ],
    )(_sc_body)
    return kern(h, r, t, ecat, rcat)


def kernel(h, r, t, batch_size, emb_e_real, emb_e_img, emb_rel_real,
           emb_rel_img):
    ecat = jnp.concatenate([emb_e_real, emb_e_img], axis=1)
    rcat = jnp.concatenate([emb_rel_real, emb_rel_img], axis=1)
    score = _run(h, r, t, ecat, rcat)
    return score[:8192], score[8192:]
